# baseline (speedup 1.0000x reference)
import functools

import jax
import jax.numpy as jnp
from jax.experimental import pallas as pl
from jax.experimental.pallas import tpu as pltpu


def _stats_kernel(x_ref, sum_ref, sq_ref):
    x = x_ref[...]
    s = jnp.sum(x, axis=2)
    q = jnp.sum(x * x, axis=2)
    sum_ref[...] = jnp.sum(s, axis=1, keepdims=True)
    sq_ref[...] = jnp.sum(q, axis=1, keepdims=True)


def _fused_kernel(x_ref, r_ref, psum_ref, psq_ref, gamma_ref, beta_ref,
                  w_ref, o_ref, *, count, eps):
    cin, ht, wd = x_ref.shape
    cout = w_ref.shape[0]
    total = jnp.sum(psum_ref[...], axis=0)
    total_sq = jnp.sum(psq_ref[...], axis=0)
    inv_count = 1.0 / count
    mean = total * inv_count
    var = total_sq * inv_count - mean * mean
    inv_std = jax.lax.rsqrt(var + eps)
    scale = (gamma_ref[...] * inv_std).reshape(cin, 1, 1)
    shift = (beta_ref[...] - mean * gamma_ref[...] * inv_std).reshape(
        cin, 1, 1)

    y = jnp.maximum(x_ref[...] * scale + shift + r_ref[...], 0.0)
    y2 = y.astype(jnp.bfloat16).reshape(cin, ht * wd)
    o = jnp.dot(w_ref[...].astype(jnp.bfloat16), y2,
                preferred_element_type=jnp.float32)
    o_ref[...] = o.reshape(cout, ht, wd)


@functools.partial(jax.jit, static_argnames=("ht",))
def _forward(x33, x26, gamma, beta, conv_w, *, ht=16):
    N, Cin, H, W = x33.shape
    Cout = conv_w.shape[0]

    psum, psq = pl.pallas_call(
        _stats_kernel,
        out_shape=(
            jax.ShapeDtypeStruct((N, Cin, 1), jnp.float32),
            jax.ShapeDtypeStruct((N, Cin, 1), jnp.float32),
        ),
        grid=(N,),
        in_specs=[pl.BlockSpec((None, Cin, H, W), lambda n: (n, 0, 0, 0))],
        out_specs=(
            pl.BlockSpec((None, Cin, 1), lambda n: (n, 0, 0)),
            pl.BlockSpec((None, Cin, 1), lambda n: (n, 0, 0)),
        ),
        compiler_params=pltpu.CompilerParams(
            dimension_semantics=("parallel",)),
    )(x33)

    w = conv_w.reshape(Cout, Cin)
    g2 = gamma.reshape(Cin, 1)
    b2 = beta.reshape(Cin, 1)

    nh = pl.cdiv(H, ht)
    out = pl.pallas_call(
        functools.partial(_fused_kernel, count=N * H * W, eps=1e-5),
        out_shape=jax.ShapeDtypeStruct((N, Cout, H, W), jnp.float32),
        grid=(N, nh),
        in_specs=[
            pl.BlockSpec((None, Cin, ht, W), lambda n, j: (n, 0, j, 0)),
            pl.BlockSpec((None, Cin, ht, W), lambda n, j: (n, 0, j, 0)),
            pl.BlockSpec((N, Cin, 1), lambda n, j: (0, 0, 0)),
            pl.BlockSpec((N, Cin, 1), lambda n, j: (0, 0, 0)),
            pl.BlockSpec((Cin, 1), lambda n, j: (0, 0)),
            pl.BlockSpec((Cin, 1), lambda n, j: (0, 0)),
            pl.BlockSpec((Cout, Cin), lambda n, j: (0, 0)),
        ],
        out_specs=pl.BlockSpec((None, Cout, ht, W), lambda n, j: (n, 0, j, 0)),
        compiler_params=pltpu.CompilerParams(
            dimension_semantics=("parallel", "parallel")),
    )(x33, x26, psum, psq, g2, b2, w)
    return out


def kernel(x33, x26, gamma, beta, conv_w):
    return _forward(x33, x26, gamma, beta, conv_w)


# speedup vs baseline: 5.5260x; 5.5260x over previous
import functools

import jax
import jax.numpy as jnp
from jax.experimental import pallas as pl
from jax.experimental.pallas import tpu as pltpu


def _stats_kernel(x_ref, sum_ref, sq_ref):
    x = x_ref[...]
    sum_ref[...] = jnp.sum(x, axis=0, keepdims=True)
    sq_ref[...] = jnp.sum(x * x, axis=0, keepdims=True)


def _fused_kernel(x_ref, r_ref, psum_ref, psq_ref, gamma_ref, beta_ref,
                  w_ref, o_ref, *, count, eps):
    total = jnp.sum(psum_ref[...], axis=0)
    total_sq = jnp.sum(psq_ref[...], axis=0)
    inv_count = 1.0 / count
    mean = total * inv_count
    var = total_sq * inv_count - mean * mean
    inv_std = jax.lax.rsqrt(var + eps)
    scale = gamma_ref[...] * inv_std
    shift = beta_ref[...] - mean * scale

    y = jnp.maximum(x_ref[...] * scale + shift + r_ref[...], 0.0)
    o_ref[...] = jax.lax.dot_general(
        y.astype(jnp.bfloat16), w_ref[...].astype(jnp.bfloat16),
        (((1,), (1,)), ((), ())),
        preferred_element_type=jnp.float32)


@functools.partial(jax.jit, static_argnames=("sblk",))
def _forward(x33, x26, gamma, beta, conv_w, *, sblk=784):
    N, Cin, H, W = x33.shape
    Cout = conv_w.shape[0]
    S = H * W

    x = x33.transpose(0, 2, 3, 1).reshape(N, S, Cin)
    r = x26.transpose(0, 2, 3, 1).reshape(N, S, Cin)

    psum, psq = pl.pallas_call(
        _stats_kernel,
        out_shape=(
            jax.ShapeDtypeStruct((N, 1, Cin), jnp.float32),
            jax.ShapeDtypeStruct((N, 1, Cin), jnp.float32),
        ),
        grid=(N,),
        in_specs=[pl.BlockSpec((None, S, Cin), lambda n: (n, 0, 0))],
        out_specs=(
            pl.BlockSpec((None, 1, Cin), lambda n: (n, 0, 0)),
            pl.BlockSpec((None, 1, Cin), lambda n: (n, 0, 0)),
        ),
        compiler_params=pltpu.CompilerParams(
            dimension_semantics=("parallel",)),
    )(x)

    w = conv_w.reshape(Cout, Cin)
    g2 = gamma.reshape(1, Cin)
    b2 = beta.reshape(1, Cin)

    ns = pl.cdiv(S, sblk)
    out = pl.pallas_call(
        functools.partial(_fused_kernel, count=N * S, eps=1e-5),
        out_shape=jax.ShapeDtypeStruct((N, S, Cout), jnp.float32),
        grid=(N, ns),
        in_specs=[
            pl.BlockSpec((None, sblk, Cin), lambda n, j: (n, j, 0)),
            pl.BlockSpec((None, sblk, Cin), lambda n, j: (n, j, 0)),
            pl.BlockSpec((N, 1, Cin), lambda n, j: (0, 0, 0)),
            pl.BlockSpec((N, 1, Cin), lambda n, j: (0, 0, 0)),
            pl.BlockSpec((1, Cin), lambda n, j: (0, 0)),
            pl.BlockSpec((1, Cin), lambda n, j: (0, 0)),
            pl.BlockSpec((Cout, Cin), lambda n, j: (0, 0)),
        ],
        out_specs=pl.BlockSpec((None, sblk, Cout), lambda n, j: (n, j, 0)),
        compiler_params=pltpu.CompilerParams(
            dimension_semantics=("parallel", "parallel")),
    )(x, r, psum, psq, g2, b2, w)
    return out.reshape(N, H, W, Cout).transpose(0, 3, 1, 2)


def kernel(x33, x26, gamma, beta, conv_w):
    return _forward(x33, x26, gamma, beta, conv_w)


# speedup vs baseline: 6.5503x; 1.1854x over previous
import functools

import jax
import jax.numpy as jnp
from jax.experimental import pallas as pl
from jax.experimental.pallas import tpu as pltpu


def _stats_kernel(x_ref, sum_ref, sq_ref):
    x = x_ref[...]
    sum_ref[...] = jnp.sum(x, axis=0, keepdims=True)
    sq_ref[...] = jnp.sum(x * x, axis=0, keepdims=True)


def _fused_kernel(x_ref, r_ref, psum_ref, psq_ref, gamma_ref, beta_ref,
                  w_ref, o_ref, *, count, eps):
    total = jnp.sum(psum_ref[...], axis=0)
    total_sq = jnp.sum(psq_ref[...], axis=0)
    inv_count = 1.0 / count
    mean = total * inv_count
    var = total_sq * inv_count - mean * mean
    inv_std = jax.lax.rsqrt(var + eps)
    scale = gamma_ref[...] * inv_std
    shift = beta_ref[...] - mean * scale

    y = jnp.maximum(x_ref[...] * scale + shift + r_ref[...], 0.0)
    o_ref[...] = jax.lax.dot_general(
        y.astype(jnp.bfloat16), w_ref[...].astype(jnp.bfloat16),
        (((1,), (1,)), ((), ())),
        preferred_element_type=jnp.float32)


@functools.partial(jax.jit, static_argnames=("sblk",))
def _forward(x33, x26, gamma, beta, conv_w, *, sblk=1568):
    N, Cin, H, W = x33.shape
    Cout = conv_w.shape[0]
    S = H * W

    x = x33.transpose(0, 2, 3, 1).reshape(N, S, Cin)
    r = x26.transpose(0, 2, 3, 1).reshape(N, S, Cin)

    psum, psq = pl.pallas_call(
        _stats_kernel,
        out_shape=(
            jax.ShapeDtypeStruct((N, 1, Cin), jnp.float32),
            jax.ShapeDtypeStruct((N, 1, Cin), jnp.float32),
        ),
        grid=(N,),
        in_specs=[pl.BlockSpec((None, S, Cin), lambda n: (n, 0, 0))],
        out_specs=(
            pl.BlockSpec((None, 1, Cin), lambda n: (n, 0, 0)),
            pl.BlockSpec((None, 1, Cin), lambda n: (n, 0, 0)),
        ),
        compiler_params=pltpu.CompilerParams(
            dimension_semantics=("parallel",)),
    )(x)

    w = conv_w.reshape(Cout, Cin)
    g2 = gamma.reshape(1, Cin)
    b2 = beta.reshape(1, Cin)

    ns = pl.cdiv(S, sblk)
    out = pl.pallas_call(
        functools.partial(_fused_kernel, count=N * S, eps=1e-5),
        out_shape=jax.ShapeDtypeStruct((N, S, Cout), jnp.float32),
        grid=(N, ns),
        in_specs=[
            pl.BlockSpec((None, sblk, Cin), lambda n, j: (n, j, 0)),
            pl.BlockSpec((None, sblk, Cin), lambda n, j: (n, j, 0)),
            pl.BlockSpec((N, 1, Cin), lambda n, j: (0, 0, 0)),
            pl.BlockSpec((N, 1, Cin), lambda n, j: (0, 0, 0)),
            pl.BlockSpec((1, Cin), lambda n, j: (0, 0)),
            pl.BlockSpec((1, Cin), lambda n, j: (0, 0)),
            pl.BlockSpec((Cout, Cin), lambda n, j: (0, 0)),
        ],
        out_specs=pl.BlockSpec((None, sblk, Cout), lambda n, j: (n, j, 0)),
        compiler_params=pltpu.CompilerParams(
            dimension_semantics=("parallel", "parallel")),
    )(x, r, psum, psq, g2, b2, w)
    return out.reshape(N, H, W, Cout).transpose(0, 3, 1, 2)


def kernel(x33, x26, gamma, beta, conv_w):
    return _forward(x33, x26, gamma, beta, conv_w)


# speedup vs baseline: 6.8061x; 1.0390x over previous
import functools

import jax
import jax.numpy as jnp
from jax.experimental import pallas as pl
from jax.experimental.pallas import tpu as pltpu


def _stats_kernel(x_ref, sum_ref, sq_ref):
    x = x_ref[...]
    sum_ref[...] = jnp.sum(x, axis=0, keepdims=True)
    sq_ref[...] = jnp.sum(x * x, axis=0, keepdims=True)


def _fused_kernel(x_ref, r_ref, psum_ref, psq_ref, gamma_ref, beta_ref,
                  w_ref, o_ref, *, count, eps):
    total = jnp.sum(psum_ref[...], axis=0)
    total_sq = jnp.sum(psq_ref[...], axis=0)
    inv_count = 1.0 / count
    mean = total * inv_count
    var = total_sq * inv_count - mean * mean
    inv_std = jax.lax.rsqrt(var + eps)
    scale = gamma_ref[...] * inv_std
    shift = beta_ref[...] - mean * scale

    y = jnp.maximum(x_ref[...] * scale + shift + r_ref[...], 0.0)
    o_ref[...] = jax.lax.dot_general(
        y.astype(jnp.bfloat16), w_ref[...].astype(jnp.bfloat16),
        (((1,), (1,)), ((), ())),
        preferred_element_type=jnp.float32)


@functools.partial(jax.jit, static_argnames=("sblk",))
def _forward(x33, x26, gamma, beta, conv_w, *, sblk=3136):
    N, Cin, H, W = x33.shape
    Cout = conv_w.shape[0]
    S = H * W

    x = x33.transpose(0, 2, 3, 1).reshape(N, S, Cin)
    r = x26.transpose(0, 2, 3, 1).reshape(N, S, Cin)

    psum, psq = pl.pallas_call(
        _stats_kernel,
        out_shape=(
            jax.ShapeDtypeStruct((N, 1, Cin), jnp.float32),
            jax.ShapeDtypeStruct((N, 1, Cin), jnp.float32),
        ),
        grid=(N,),
        in_specs=[pl.BlockSpec((None, S, Cin), lambda n: (n, 0, 0))],
        out_specs=(
            pl.BlockSpec((None, 1, Cin), lambda n: (n, 0, 0)),
            pl.BlockSpec((None, 1, Cin), lambda n: (n, 0, 0)),
        ),
        compiler_params=pltpu.CompilerParams(
            dimension_semantics=("parallel",)),
    )(x)

    w = conv_w.reshape(Cout, Cin)
    g2 = gamma.reshape(1, Cin)
    b2 = beta.reshape(1, Cin)

    ns = pl.cdiv(S, sblk)
    out = pl.pallas_call(
        functools.partial(_fused_kernel, count=N * S, eps=1e-5),
        out_shape=jax.ShapeDtypeStruct((N, S, Cout), jnp.float32),
        grid=(N, ns),
        in_specs=[
            pl.BlockSpec((None, sblk, Cin), lambda n, j: (n, j, 0)),
            pl.BlockSpec((None, sblk, Cin), lambda n, j: (n, j, 0)),
            pl.BlockSpec((N, 1, Cin), lambda n, j: (0, 0, 0)),
            pl.BlockSpec((N, 1, Cin), lambda n, j: (0, 0, 0)),
            pl.BlockSpec((1, Cin), lambda n, j: (0, 0)),
            pl.BlockSpec((1, Cin), lambda n, j: (0, 0)),
            pl.BlockSpec((Cout, Cin), lambda n, j: (0, 0)),
        ],
        out_specs=pl.BlockSpec((None, sblk, Cout), lambda n, j: (n, j, 0)),
        compiler_params=pltpu.CompilerParams(
            dimension_semantics=("parallel", "parallel")),
    )(x, r, psum, psq, g2, b2, w)
    return out.reshape(N, H, W, Cout).transpose(0, 3, 1, 2)


def kernel(x33, x26, gamma, beta, conv_w):
    return _forward(x33, x26, gamma, beta, conv_w)
